# baseline (speedup 1.0000x reference)
import functools
import math

import jax
import jax.numpy as jnp
from jax.experimental import pallas as pl

EMBED = 1024
HEADS = 16
HEAD_DIM = 64
SCALE = HEAD_DIM ** -0.5
WINDOW = 2
FAR = 4
S = 2048
BQ = 256

_LOG1E4 = math.log(10000.0)


def _qkv_rope_kernel(x_ref, w_ref, b_ref, o_ref):
    qi = pl.program_id(0)
    y = jnp.dot(x_ref[...], w_ref[...], preferred_element_type=jnp.float32)
    y = y + b_ref[...]
    bq, n = y.shape
    y_p32 = jnp.concatenate([y[:, 32:], y[:, :32]], axis=1)
    y_m32 = jnp.concatenate([y[:, -32:], y[:, :-32]], axis=1)
    col = jax.lax.broadcasted_iota(jnp.int32, (bq, n), 1)
    d = col % HEAD_DIM
    dr = d % (HEAD_DIM // 2)
    hi = d >= (HEAD_DIM // 2)
    partner = jnp.where(hi, y_m32, y_p32)
    inv = jnp.exp(dr.astype(jnp.float32) * (-_LOG1E4 / (HEAD_DIM // 2)))
    row = jax.lax.broadcasted_iota(jnp.int32, (bq, n), 0)
    pos = (qi * bq + row).astype(jnp.float32)
    ang = pos * inv
    c = jnp.cos(ang)
    s = jnp.sin(ang)
    roped = y * c + partner * jnp.where(hi, s, -s)
    o_ref[...] = jnp.where(col < 2 * EMBED, roped, y).astype(jnp.bfloat16)


def _attn_kernel(q2_ref, k2_ref, v2_ref, barc_ref, barr_ref, instc_ref,
                 instr_ref, o_ref):
    qi = pl.program_id(1)
    bq = q2_ref.shape[0]
    s = k2_ref.shape[0]
    i = qi * bq + jax.lax.broadcasted_iota(jnp.int32, (bq, s), 0)
    j = jax.lax.broadcasted_iota(jnp.int32, (bq, s), 1)
    causal = j <= i
    bar_q = barc_ref[...]
    bar_k = barr_ref[...]
    inst_q = instc_ref[...]
    inst_k = instr_ref[...]

    same = (inst_q == inst_k) & (inst_q < 129) & causal
    off = bar_q - bar_k
    nearfar = ((off >= 0) & (off <= WINDOW)) | (off == FAR)
    cross = ((inst_q < 129) & (bar_q >= 0) & (inst_k != inst_q)
             & (inst_k < 129) & nearfar)
    glob = ((inst_k == 129) | (bar_k == -1)) & causal

    outs = []
    for t in range(2):
        q = q2_ref[:, t * HEAD_DIM:(t + 1) * HEAD_DIM]
        k = k2_ref[:, t * HEAD_DIM:(t + 1) * HEAD_DIM]
        v = v2_ref[:, t * HEAD_DIM:(t + 1) * HEAD_DIM]
        scores = jax.lax.dot_general(
            q, k, (((1,), (1,)), ((), ())),
            preferred_element_type=jnp.float32) * SCALE
        acc = None
        for mask in (same, cross, glob):
            sc = jnp.where(mask, scores, -1e30)
            m = jnp.max(sc, axis=-1, keepdims=True)
            e = jnp.where(mask, jnp.exp(sc - m), 0.0)
            ssum = jnp.sum(e, axis=-1, keepdims=True)
            attn = (e / jnp.where(ssum == 0.0, 1.0, ssum)).astype(jnp.bfloat16)
            out = jnp.dot(attn, v, preferred_element_type=jnp.float32)
            acc = out if acc is None else acc + out
        outs.append(acc)
    o_ref[...] = jnp.concatenate(outs, axis=1).astype(jnp.bfloat16)


def _out_proj_kernel(a_ref, w_ref, b_ref, o_ref):
    o_ref[...] = jnp.dot(a_ref[...], w_ref[...],
                         preferred_element_type=jnp.float32) + b_ref[...]


@jax.jit
def kernel(x, bar_ids, instrument_ids, Wq, bq, Wk, bk, Wv, bv, Wo, bo):
    B, s, e = x.shape
    x2 = x.reshape(s, e).astype(jnp.bfloat16)
    Wqkv = jnp.concatenate([Wq.T, Wk.T, Wv.T], axis=1).astype(jnp.bfloat16)
    bqkv = jnp.concatenate([bq, bk, bv]).reshape(1, 3 * e)

    nq = s // BQ
    qkv = pl.pallas_call(
        _qkv_rope_kernel,
        grid=(nq,),
        in_specs=[
            pl.BlockSpec((BQ, e), lambda i: (i, 0)),
            pl.BlockSpec((e, 3 * e), lambda i: (0, 0)),
            pl.BlockSpec((1, 3 * e), lambda i: (0, 0)),
        ],
        out_specs=pl.BlockSpec((BQ, 3 * e), lambda i: (i, 0)),
        out_shape=jax.ShapeDtypeStruct((s, 3 * e), jnp.bfloat16),
    )(x2, Wqkv, bqkv)

    bar_c = bar_ids.reshape(s, 1)
    bar_r = bar_ids.reshape(1, s)
    inst_c = instrument_ids.reshape(s, 1)
    inst_r = instrument_ids.reshape(1, s)

    attn = pl.pallas_call(
        _attn_kernel,
        grid=(HEADS // 2, nq),
        in_specs=[
            pl.BlockSpec((BQ, 2 * HEAD_DIM), lambda h, i: (i, h)),
            pl.BlockSpec((S, 2 * HEAD_DIM), lambda h, i: (0, 8 + h)),
            pl.BlockSpec((S, 2 * HEAD_DIM), lambda h, i: (0, 16 + h)),
            pl.BlockSpec((BQ, 1), lambda h, i: (i, 0)),
            pl.BlockSpec((1, S), lambda h, i: (0, 0)),
            pl.BlockSpec((BQ, 1), lambda h, i: (i, 0)),
            pl.BlockSpec((1, S), lambda h, i: (0, 0)),
        ],
        out_specs=pl.BlockSpec((BQ, 2 * HEAD_DIM), lambda h, i: (i, h)),
        out_shape=jax.ShapeDtypeStruct((s, e), jnp.bfloat16),
    )(qkv, qkv, qkv, bar_c, bar_r, inst_c, inst_r)

    out = pl.pallas_call(
        _out_proj_kernel,
        grid=(nq,),
        in_specs=[
            pl.BlockSpec((BQ, e), lambda i: (i, 0)),
            pl.BlockSpec((e, e), lambda i: (0, 0)),
            pl.BlockSpec((1, e), lambda i: (0, 0)),
        ],
        out_specs=pl.BlockSpec((BQ, e), lambda i: (i, 0)),
        out_shape=jax.ShapeDtypeStruct((s, e), jnp.float32),
    )(attn, Wo.T.astype(jnp.bfloat16), bo.reshape(1, e))

    return out.reshape(B, s, e)


# speedup vs baseline: 1.1526x; 1.1526x over previous
import functools
import math

import jax
import jax.numpy as jnp
from jax.experimental import pallas as pl

EMBED = 1024
HEADS = 16
HEAD_DIM = 64
SCALE = HEAD_DIM ** -0.5
WINDOW = 2
FAR = 4
S = 2048
BQ = 256

_LOG1E4 = math.log(10000.0)


def _qkv_rope_kernel(x_ref, w_ref, b_ref, o_ref):
    qi = pl.program_id(0)
    y = jnp.dot(x_ref[...], w_ref[...], preferred_element_type=jnp.float32)
    y = y + b_ref[...]
    bq, n = y.shape
    y_p32 = jnp.concatenate([y[:, 32:], y[:, :32]], axis=1)
    y_m32 = jnp.concatenate([y[:, -32:], y[:, :-32]], axis=1)
    col = jax.lax.broadcasted_iota(jnp.int32, (bq, n), 1)
    d = col % HEAD_DIM
    dr = d % (HEAD_DIM // 2)
    hi = d >= (HEAD_DIM // 2)
    partner = jnp.where(hi, y_m32, y_p32)
    inv = jnp.exp(dr.astype(jnp.float32) * (-_LOG1E4 / (HEAD_DIM // 2)))
    row = jax.lax.broadcasted_iota(jnp.int32, (bq, n), 0)
    pos = (qi * bq + row).astype(jnp.float32)
    ang = pos * inv
    c = jnp.cos(ang)
    s = jnp.sin(ang)
    roped = y * c + partner * jnp.where(hi, s, -s)
    o_ref[...] = jnp.where(col < 2 * EMBED, roped, y).astype(jnp.bfloat16)


def _attn_kernel(q2_ref, k2_ref, v2_ref, barc_ref, barr_ref, instc_ref,
                 instr_ref, o_ref):
    qi = pl.program_id(1)
    bq = q2_ref.shape[0]
    s = k2_ref.shape[0]
    i = qi * bq + jax.lax.broadcasted_iota(jnp.int32, (bq, s), 0)
    j = jax.lax.broadcasted_iota(jnp.int32, (bq, s), 1)
    causal = j <= i
    bar_q = barc_ref[...]
    bar_k = barr_ref[...]
    inst_q = instc_ref[...]
    inst_k = instr_ref[...]

    same = (inst_q == inst_k) & (inst_q < 129) & causal
    off = bar_q - bar_k
    nearfar = ((off >= 0) & (off <= WINDOW)) | (off == FAR)
    cross = ((inst_q < 129) & (bar_q >= 0) & (inst_k != inst_q)
             & (inst_k < 129) & nearfar)
    glob = ((inst_k == 129) | (bar_k == -1)) & causal

    union = same | cross | glob
    outs = []
    for t in range(2):
        q = q2_ref[:, t * HEAD_DIM:(t + 1) * HEAD_DIM]
        k = k2_ref[:, t * HEAD_DIM:(t + 1) * HEAD_DIM]
        v = v2_ref[:, t * HEAD_DIM:(t + 1) * HEAD_DIM]
        scores = jax.lax.dot_general(
            q, k, (((1,), (1,)), ((), ())),
            preferred_element_type=jnp.float32) * SCALE
        sc = jnp.where(union, scores, -1e30)
        m = jnp.max(sc, axis=-1, keepdims=True)
        e = jnp.exp(sc - m)
        invs = []
        for mask in (same, cross, glob):
            ssum = jnp.sum(jnp.where(mask, e, 0.0), axis=-1, keepdims=True)
            invs.append(1.0 / jnp.where(ssum == 0.0, 1.0, ssum))
        denom = jnp.where(same, invs[0],
                          jnp.where(cross, invs[1],
                                    jnp.where(glob, invs[2], 0.0)))
        w = (e * denom).astype(jnp.bfloat16)
        outs.append(jnp.dot(w, v, preferred_element_type=jnp.float32))
    o_ref[...] = jnp.concatenate(outs, axis=1).astype(jnp.bfloat16)


def _out_proj_kernel(a_ref, w_ref, b_ref, o_ref):
    o_ref[...] = jnp.dot(a_ref[...], w_ref[...],
                         preferred_element_type=jnp.float32) + b_ref[...]


@jax.jit
def kernel(x, bar_ids, instrument_ids, Wq, bq, Wk, bk, Wv, bv, Wo, bo):
    B, s, e = x.shape
    x2 = x.reshape(s, e).astype(jnp.bfloat16)
    Wqkv = jnp.concatenate([Wq.T, Wk.T, Wv.T], axis=1).astype(jnp.bfloat16)
    bqkv = jnp.concatenate([bq, bk, bv]).reshape(1, 3 * e)

    nq = s // BQ
    qkv = pl.pallas_call(
        _qkv_rope_kernel,
        grid=(nq,),
        in_specs=[
            pl.BlockSpec((BQ, e), lambda i: (i, 0)),
            pl.BlockSpec((e, 3 * e), lambda i: (0, 0)),
            pl.BlockSpec((1, 3 * e), lambda i: (0, 0)),
        ],
        out_specs=pl.BlockSpec((BQ, 3 * e), lambda i: (i, 0)),
        out_shape=jax.ShapeDtypeStruct((s, 3 * e), jnp.bfloat16),
    )(x2, Wqkv, bqkv)

    bar_c = bar_ids.reshape(s, 1)
    bar_r = bar_ids.reshape(1, s)
    inst_c = instrument_ids.reshape(s, 1)
    inst_r = instrument_ids.reshape(1, s)

    attn = pl.pallas_call(
        _attn_kernel,
        grid=(HEADS // 2, nq),
        in_specs=[
            pl.BlockSpec((BQ, 2 * HEAD_DIM), lambda h, i: (i, h)),
            pl.BlockSpec((S, 2 * HEAD_DIM), lambda h, i: (0, 8 + h)),
            pl.BlockSpec((S, 2 * HEAD_DIM), lambda h, i: (0, 16 + h)),
            pl.BlockSpec((BQ, 1), lambda h, i: (i, 0)),
            pl.BlockSpec((1, S), lambda h, i: (0, 0)),
            pl.BlockSpec((BQ, 1), lambda h, i: (i, 0)),
            pl.BlockSpec((1, S), lambda h, i: (0, 0)),
        ],
        out_specs=pl.BlockSpec((BQ, 2 * HEAD_DIM), lambda h, i: (i, h)),
        out_shape=jax.ShapeDtypeStruct((s, e), jnp.bfloat16),
    )(qkv, qkv, qkv, bar_c, bar_r, inst_c, inst_r)

    out = pl.pallas_call(
        _out_proj_kernel,
        grid=(nq,),
        in_specs=[
            pl.BlockSpec((BQ, e), lambda i: (i, 0)),
            pl.BlockSpec((e, e), lambda i: (0, 0)),
            pl.BlockSpec((1, e), lambda i: (0, 0)),
        ],
        out_specs=pl.BlockSpec((BQ, e), lambda i: (i, 0)),
        out_shape=jax.ShapeDtypeStruct((s, e), jnp.float32),
    )(attn, Wo.T.astype(jnp.bfloat16), bo.reshape(1, e))

    return out.reshape(B, s, e)


# speedup vs baseline: 1.6699x; 1.4488x over previous
import functools
import math

import jax
import jax.numpy as jnp
from jax.experimental import pallas as pl
from jax.experimental.pallas import tpu as pltpu

EMBED = 1024
HEADS = 16
HEAD_DIM = 64
SCALE = HEAD_DIM ** -0.5
WINDOW = 2
FAR = 4
S = 2048
BQ = 256

_LOG1E4 = math.log(10000.0)


def _qkv_rope_kernel(x_ref, w_ref, b_ref, o_ref):
    qi = pl.program_id(0)
    y = jnp.dot(x_ref[...], w_ref[...], preferred_element_type=jnp.float32)
    y = y + b_ref[...]
    bq, n = y.shape
    y_p32 = jnp.concatenate([y[:, 32:], y[:, :32]], axis=1)
    y_m32 = jnp.concatenate([y[:, -32:], y[:, :-32]], axis=1)
    col = jax.lax.broadcasted_iota(jnp.int32, (bq, n), 1)
    d = col % HEAD_DIM
    dr = d % (HEAD_DIM // 2)
    hi = d >= (HEAD_DIM // 2)
    partner = jnp.where(hi, y_m32, y_p32)
    inv = jnp.exp(dr.astype(jnp.float32) * (-_LOG1E4 / (HEAD_DIM // 2)))
    row = jax.lax.broadcasted_iota(jnp.int32, (bq, n), 0)
    pos = (qi * bq + row).astype(jnp.float32)
    ang = pos * inv
    c = jnp.cos(ang)
    s = jnp.sin(ang)
    roped = y * c + partner * jnp.where(hi, s, -s)
    out = jnp.where(col < 2 * EMBED, roped, y)
    out = out * jnp.where(col < EMBED, SCALE, 1.0)
    o_ref[...] = out.astype(jnp.bfloat16)


def _attn_kernel(barc_ref, barr_ref, instc_ref, instr_ref, qkv_ref, o_ref,
                 bias_ref, ms_ref, mc_ref, mg_ref):
    qi = pl.program_id(0)
    hp = pl.program_id(1)
    s = qkv_ref.shape[0]

    @pl.when(hp == 0)
    def _build_masks():
        i = qi * BQ + jax.lax.broadcasted_iota(jnp.int32, (BQ, s), 0)
        j = jax.lax.broadcasted_iota(jnp.int32, (BQ, s), 1)
        causal = j <= i
        bar_q = barc_ref[...]
        bar_k = barr_ref[...]
        inst_q = instc_ref[...]
        inst_k = instr_ref[...]
        same = (inst_q == inst_k) & (inst_q < 129) & causal
        off = bar_q - bar_k
        nearfar = ((off >= 0) & (off <= WINDOW)) | (off == FAR)
        cross = ((inst_q < 129) & (bar_q >= 0) & (inst_k != inst_q)
                 & (inst_k < 129) & nearfar)
        glob = ((inst_k == 129) | (bar_k == -1)) & causal
        union = same | cross | glob
        ms_ref[...] = same.astype(jnp.float32)
        mc_ref[...] = cross.astype(jnp.float32)
        mg_ref[...] = glob.astype(jnp.float32)
        bias_ref[...] = jnp.where(union, 0.0, -1e30)

    q2 = qkv_ref[pl.ds(qi * BQ, BQ), pl.ds(hp * 128, 128)]
    k2 = qkv_ref[:, pl.ds(EMBED + hp * 128, 128)]
    v2 = qkv_ref[:, pl.ds(2 * EMBED + hp * 128, 128)]
    bias = bias_ref[...]
    ms = ms_ref[...]
    mc = mc_ref[...]
    mg = mg_ref[...]
    outs = []
    for t in range(2):
        q = q2[:, t * HEAD_DIM:(t + 1) * HEAD_DIM]
        k = k2[:, t * HEAD_DIM:(t + 1) * HEAD_DIM]
        v = v2[:, t * HEAD_DIM:(t + 1) * HEAD_DIM]
        scores = jax.lax.dot_general(
            q, k, (((1,), (1,)), ((), ())),
            preferred_element_type=jnp.float32) + bias
        m = jnp.max(scores, axis=-1, keepdims=True)
        e = jnp.exp(scores - m)
        invs = []
        for mask in (ms, mc, mg):
            ssum = jnp.sum(e * mask, axis=-1, keepdims=True)
            invs.append(1.0 / jnp.where(ssum == 0.0, 1.0, ssum))
        denom = ms * invs[0] + mc * invs[1] + mg * invs[2]
        w = (e * denom).astype(jnp.bfloat16)
        outs.append(jnp.dot(w, v, preferred_element_type=jnp.float32))
    o_ref[...] = jnp.concatenate(outs, axis=1).astype(jnp.bfloat16)


def _out_proj_kernel(a_ref, w_ref, b_ref, o_ref):
    o_ref[...] = jnp.dot(a_ref[...], w_ref[...],
                         preferred_element_type=jnp.float32) + b_ref[...]


@jax.jit
def kernel(x, bar_ids, instrument_ids, Wq, bq, Wk, bk, Wv, bv, Wo, bo):
    B, s, e = x.shape
    x2 = x.reshape(s, e).astype(jnp.bfloat16)
    Wqkv = jnp.concatenate([Wq.T, Wk.T, Wv.T], axis=1).astype(jnp.bfloat16)
    bqkv = jnp.concatenate([bq, bk, bv]).reshape(1, 3 * e)

    nq = s // BQ
    qkv = pl.pallas_call(
        _qkv_rope_kernel,
        grid=(nq,),
        in_specs=[
            pl.BlockSpec((BQ, e), lambda i: (i, 0)),
            pl.BlockSpec((e, 3 * e), lambda i: (0, 0)),
            pl.BlockSpec((1, 3 * e), lambda i: (0, 0)),
        ],
        out_specs=pl.BlockSpec((BQ, 3 * e), lambda i: (i, 0)),
        out_shape=jax.ShapeDtypeStruct((s, 3 * e), jnp.bfloat16),
    )(x2, Wqkv, bqkv)

    bar_c = bar_ids.reshape(s, 1)
    bar_r = bar_ids.reshape(1, s)
    inst_c = instrument_ids.reshape(s, 1)
    inst_r = instrument_ids.reshape(1, s)

    attn = pl.pallas_call(
        _attn_kernel,
        grid=(nq, HEADS // 2),
        in_specs=[
            pl.BlockSpec((BQ, 1), lambda i, h: (i, 0)),
            pl.BlockSpec((1, S), lambda i, h: (0, 0)),
            pl.BlockSpec((BQ, 1), lambda i, h: (i, 0)),
            pl.BlockSpec((1, S), lambda i, h: (0, 0)),
            pl.BlockSpec((S, 3 * EMBED), lambda i, h: (0, 0)),
        ],
        out_specs=pl.BlockSpec((BQ, 2 * HEAD_DIM), lambda i, h: (i, h)),
        out_shape=jax.ShapeDtypeStruct((s, e), jnp.bfloat16),
        scratch_shapes=[
            pltpu.VMEM((BQ, S), jnp.float32),
            pltpu.VMEM((BQ, S), jnp.float32),
            pltpu.VMEM((BQ, S), jnp.float32),
            pltpu.VMEM((BQ, S), jnp.float32),
        ],
    )(bar_c, bar_r, inst_c, inst_r, qkv)

    out = pl.pallas_call(
        _out_proj_kernel,
        grid=(nq,),
        in_specs=[
            pl.BlockSpec((BQ, e), lambda i: (i, 0)),
            pl.BlockSpec((e, e), lambda i: (0, 0)),
            pl.BlockSpec((1, e), lambda i: (0, 0)),
        ],
        out_specs=pl.BlockSpec((BQ, e), lambda i: (i, 0)),
        out_shape=jax.ShapeDtypeStruct((s, e), jnp.float32),
    )(attn, Wo.T.astype(jnp.bfloat16), bo.reshape(1, e))

    return out.reshape(B, s, e)
